# baseline (speedup 1.0000x reference)
import functools

import jax
import jax.numpy as jnp
from jax import lax
from jax.experimental import pallas as pl
from jax.experimental.pallas import tpu as pltpu
from jax.experimental.pallas import tpu_sc as plsc

N = 4096 * 200
D = 64
NC = 2
NS = 16
NW = NC * NS
PER_W = N // NW
CHUNK = 128
NCHUNK = PER_W // CHUNK

_mesh = plsc.VectorSubcoreMesh(core_axis_name="c", subcore_axis_name="s")


@functools.partial(
    pl.kernel,
    mesh=_mesh,
    out_type=jax.ShapeDtypeStruct((N, D), jnp.float32),
    scratch_types=[
        pltpu.VMEM((CHUNK,), jnp.int32),
        pltpu.VMEM((CHUNK, D), jnp.float32),
        pltpu.SemaphoreType.DMA,
    ],
    compiler_params=pltpu.CompilerParams(use_tc_tiling_on_sc=False),
)
def _emb_lookup(idx_hbm, table_hbm, out_hbm, idx_v, rows_v, sem):
    wid = lax.axis_index("s") * NC + lax.axis_index("c")
    base = wid * PER_W

    def body(g, carry):
        off = base + g * CHUNK
        pltpu.sync_copy(idx_hbm.at[pl.ds(off, CHUNK)], idx_v)
        pltpu.async_copy(table_hbm.at[idx_v], rows_v, sem).wait()
        pltpu.sync_copy(rows_v, out_hbm.at[pl.ds(off, CHUNK)])
        return carry

    lax.fori_loop(0, NCHUNK, body, 0)


def kernel(input_ids, weight):
    idx = input_ids.reshape(-1).astype(jnp.int32)
    out = _emb_lookup(idx, weight)
    return out.reshape(input_ids.shape + (weight.shape[1],))


# speedup vs baseline: 1.1948x; 1.1948x over previous
import functools

import jax
import jax.numpy as jnp
from jax import lax
from jax.experimental import pallas as pl
from jax.experimental.pallas import tpu as pltpu
from jax.experimental.pallas import tpu_sc as plsc

N = 4096 * 200
D = 64
NC = 2
NS = 16
NW = NC * NS
CHUNK = 128
NCH = N // CHUNK
CPW = NCH // NW
K = 4
NSUP = CPW // K
T_STEADY = NSUP // 3

_mesh = plsc.VectorSubcoreMesh(core_axis_name="c", subcore_axis_name="s")


@functools.partial(
    pl.kernel,
    mesh=_mesh,
    out_type=jax.ShapeDtypeStruct((NCH, CHUNK, D), jnp.float32),
    scratch_types=[
        pltpu.VMEM((CPW, CHUNK), jnp.int32),
        pltpu.VMEM((K, CHUNK, D), jnp.float32),
        pltpu.VMEM((K, CHUNK, D), jnp.float32),
        pltpu.VMEM((K, CHUNK, D), jnp.float32),
        pltpu.SemaphoreType.DMA,
        pltpu.SemaphoreType.DMA,
        pltpu.SemaphoreType.DMA,
        pltpu.SemaphoreType.DMA,
        pltpu.SemaphoreType.DMA,
        pltpu.SemaphoreType.DMA,
    ],
    compiler_params=pltpu.CompilerParams(use_tc_tiling_on_sc=False),
)
def _emb_lookup(idx_hbm, table_hbm, out_hbm, idx_all, rows0, rows1, rows2,
                g0, g1, g2, s0, s1, s2):
    rows = (rows0, rows1, rows2)
    gsem = (g0, g1, g2)
    ssem = (s0, s1, s2)
    wid = lax.axis_index("s") * NC + lax.axis_index("c")
    base = wid * CPW

    def fire(g, j):
        for k in range(K):
            pltpu.async_copy(table_hbm.at[idx_all.at[g * K + k]],
                             rows[j].at[k], gsem[j])

    def wait_fire(j):
        pltpu.make_async_copy(out_hbm.at[pl.ds(0, K)], rows[j], gsem[j]).wait()

    def store(g, j):
        pltpu.async_copy(rows[j], out_hbm.at[pl.ds(base + g * K, K)], ssem[j])

    def wait_store(j):
        pltpu.make_async_copy(rows[j], out_hbm.at[pl.ds(0, K)], ssem[j]).wait()

    pltpu.sync_copy(idx_hbm.at[pl.ds(base, CPW)], idx_all)

    fire(0, 0)
    fire(1, 1)
    fire(2, 2)
    wait_fire(0)
    store(0, 0)
    wait_fire(1)
    store(1, 1)

    def body(t, carry):
        for j in range(3):
            g = 3 * t + j
            p = (j + 2) % 3
            wait_store(j)
            fire(g, j)
            wait_fire(p)
            store(g - 1, p)
        return carry

    lax.fori_loop(1, T_STEADY, body, 0)

    wait_store(0)
    fire(48, 0)
    wait_fire(2)
    store(47, 2)
    wait_store(1)
    fire(49, 1)
    wait_fire(0)
    store(48, 0)
    wait_fire(1)
    store(49, 1)
    wait_store(0)
    wait_store(1)
    wait_store(2)


def kernel(input_ids, weight):
    idx = input_ids.reshape(NCH, CHUNK).astype(jnp.int32)
    out = _emb_lookup(idx, weight)
    return out.reshape(input_ids.shape + (weight.shape[1],))


# speedup vs baseline: 1.1965x; 1.0014x over previous
import functools

import jax
import jax.numpy as jnp
from jax import lax
from jax.experimental import pallas as pl
from jax.experimental.pallas import tpu as pltpu
from jax.experimental.pallas import tpu_sc as plsc

B = 4096
S = 200
D = 64
NC = 2
NS = 16
NW = NC * NS
RPW = B // NW
K = 2
NSUP = RPW // K
T_STEADY = 21

_mesh = plsc.VectorSubcoreMesh(core_axis_name="c", subcore_axis_name="s")


@functools.partial(
    pl.kernel,
    mesh=_mesh,
    out_type=jax.ShapeDtypeStruct((B, S, D), jnp.float32),
    scratch_types=[
        pltpu.VMEM((RPW, S), jnp.int32),
        pltpu.VMEM((K, S, D), jnp.float32),
        pltpu.VMEM((K, S, D), jnp.float32),
        pltpu.VMEM((K, S, D), jnp.float32),
        pltpu.SemaphoreType.DMA,
        pltpu.SemaphoreType.DMA,
        pltpu.SemaphoreType.DMA,
        pltpu.SemaphoreType.DMA,
        pltpu.SemaphoreType.DMA,
        pltpu.SemaphoreType.DMA,
    ],
    compiler_params=pltpu.CompilerParams(use_tc_tiling_on_sc=False),
)
def _emb_lookup(idx_hbm, table_hbm, out_hbm, idx_all, rows0, rows1, rows2,
                g0, g1, g2, s0, s1, s2):
    rows = (rows0, rows1, rows2)
    gsem = (g0, g1, g2)
    ssem = (s0, s1, s2)
    wid = lax.axis_index("s") * NC + lax.axis_index("c")
    base = wid * RPW

    def fire(g, j):
        for k in range(K):
            r = g * K + k
            pltpu.async_copy(table_hbm.at[idx_all.at[r, pl.ds(0, 128)]],
                             rows[j].at[k, pl.ds(0, 128)], gsem[j])
            pltpu.async_copy(table_hbm.at[idx_all.at[r, pl.ds(128, S - 128)]],
                             rows[j].at[k, pl.ds(128, S - 128)], gsem[j])

    def wait_fire(j):
        pltpu.make_async_copy(out_hbm.at[pl.ds(0, K)], rows[j], gsem[j]).wait()

    def store(g, j):
        pltpu.async_copy(rows[j], out_hbm.at[pl.ds(base + g * K, K)], ssem[j])

    def wait_store(j):
        pltpu.make_async_copy(rows[j], out_hbm.at[pl.ds(0, K)], ssem[j]).wait()

    pltpu.sync_copy(idx_hbm.at[pl.ds(base, RPW)], idx_all)

    fire(0, 0)
    fire(1, 1)
    fire(2, 2)
    wait_fire(0)
    store(0, 0)
    wait_fire(1)
    store(1, 1)

    def body(t, carry):
        for j in range(3):
            g = 3 * t + j
            p = (j + 2) % 3
            wait_store(j)
            fire(g, j)
            wait_fire(p)
            store(g - 1, p)
        return carry

    lax.fori_loop(1, T_STEADY, body, 0)

    wait_store(0)
    fire(63, 0)
    wait_fire(2)
    store(62, 2)
    wait_fire(0)
    store(63, 0)
    wait_store(0)
    wait_store(1)
    wait_store(2)


def kernel(input_ids, weight):
    return _emb_lookup(input_ids.astype(jnp.int32), weight)


# speedup vs baseline: 1.4589x; 1.2194x over previous
import functools

import jax
import jax.numpy as jnp
from jax import lax
from jax.experimental import pallas as pl
from jax.experimental.pallas import tpu as pltpu
from jax.experimental.pallas import tpu_sc as plsc

B = 4096
S = 200
D = 64
DP = 128
NC = 2
NS = 16
NW = NC * NS
RPW = B // NW
NSUP = RPW
T_STEADY = 42

_mesh = plsc.VectorSubcoreMesh(core_axis_name="c", subcore_axis_name="s")


@functools.partial(
    pl.kernel,
    mesh=_mesh,
    out_type=jax.ShapeDtypeStruct((B, S, DP), jnp.float32),
    scratch_types=[
        pltpu.VMEM((RPW, S), jnp.int32),
        pltpu.VMEM((S, DP), jnp.float32),
        pltpu.VMEM((S, DP), jnp.float32),
        pltpu.VMEM((S, DP), jnp.float32),
        pltpu.SemaphoreType.DMA,
        pltpu.SemaphoreType.DMA,
        pltpu.SemaphoreType.DMA,
        pltpu.SemaphoreType.DMA,
        pltpu.SemaphoreType.DMA,
        pltpu.SemaphoreType.DMA,
    ],
    compiler_params=pltpu.CompilerParams(use_tc_tiling_on_sc=False),
)
def _emb_lookup(idx_hbm, table_hbm, out_hbm, idx_all, rows0, rows1, rows2,
                g0, g1, g2, s0, s1, s2):
    rows = (rows0, rows1, rows2)
    gsem = (g0, g1, g2)
    ssem = (s0, s1, s2)
    wid = lax.axis_index("s") * NC + lax.axis_index("c")
    base = wid * RPW

    def fire(g, j):
        pltpu.async_copy(table_hbm.at[idx_all.at[g, pl.ds(0, 128)]],
                         rows[j].at[pl.ds(0, 128)], gsem[j])
        pltpu.async_copy(table_hbm.at[idx_all.at[g, pl.ds(128, S - 128)]],
                         rows[j].at[pl.ds(128, S - 128)], gsem[j])

    def wait_fire(j):
        pltpu.make_async_copy(out_hbm.at[0], rows[j], gsem[j]).wait()

    def store(g, j):
        pltpu.async_copy(rows[j], out_hbm.at[base + g], ssem[j])

    def wait_store(j):
        pltpu.make_async_copy(rows[j], out_hbm.at[0], ssem[j]).wait()

    pltpu.sync_copy(idx_hbm.at[pl.ds(base, RPW)], idx_all)

    fire(0, 0)
    fire(1, 1)
    fire(2, 2)
    wait_fire(0)
    store(0, 0)
    wait_fire(1)
    store(1, 1)

    def body(t, carry):
        for j in range(3):
            g = 3 * t + j
            p = (j + 2) % 3
            wait_store(j)
            fire(g, j)
            wait_fire(p)
            store(g - 1, p)
        return carry

    lax.fori_loop(1, T_STEADY, body, 0)

    wait_store(0)
    fire(126, 0)
    wait_fire(2)
    store(125, 2)
    wait_store(1)
    fire(127, 1)
    wait_fire(0)
    store(126, 0)
    wait_fire(1)
    store(127, 1)
    wait_store(0)
    wait_store(1)
    wait_store(2)


def kernel(input_ids, weight):
    wt = jnp.pad(weight, ((0, 0), (0, DP - D)))
    out = _emb_lookup(input_ids.astype(jnp.int32), wt)
    return out[..., :D]


# speedup vs baseline: 1.5919x; 1.0911x over previous
import functools

import jax
import jax.numpy as jnp
from jax import lax
from jax.experimental import pallas as pl
from jax.experimental.pallas import tpu as pltpu
from jax.experimental.pallas import tpu_sc as plsc

B = 4096
S = 200
D = 64
NC = 2
NS = 16
NW = NC * NS
RPW = B // NW
T_STEADY = 42

_mesh = plsc.VectorSubcoreMesh(core_axis_name="c", subcore_axis_name="s")


@functools.partial(
    pl.kernel,
    mesh=_mesh,
    out_type=jax.ShapeDtypeStruct((B, S, D), jnp.float32),
    scratch_types=[
        pltpu.VMEM((RPW, S), jnp.int32),
        pltpu.VMEM((S, D), jnp.float32),
        pltpu.VMEM((S, D), jnp.float32),
        pltpu.VMEM((S, D), jnp.float32),
        pltpu.SemaphoreType.DMA,
        pltpu.SemaphoreType.DMA,
        pltpu.SemaphoreType.DMA,
        pltpu.SemaphoreType.DMA,
        pltpu.SemaphoreType.DMA,
        pltpu.SemaphoreType.DMA,
    ],
    compiler_params=pltpu.CompilerParams(use_tc_tiling_on_sc=True),
)
def _emb_lookup(idx_hbm, table_hbm, out_hbm, idx_all,
                rows0, rows1, rows2, g0, g1, g2, s0, s1, s2):
    rows = (rows0, rows1, rows2)
    gsem = (g0, g1, g2)
    ssem = (s0, s1, s2)
    wid = lax.axis_index("s") * NC + lax.axis_index("c")
    base = wid * RPW

    def fire(g, j):
        def enq(vec, pos, lanes):
            for l in lanes:
                v = vec[l]
                pltpu.async_copy(table_hbm.at[v], rows[j].at[pos + l],
                                 gsem[j])

        def chunk(c, carry):
            enq(idx_all[g, pl.ds(c * 16, 16)], c * 16, range(16))
            return carry

        lax.fori_loop(0, 12, chunk, 0)
        enq(idx_all[g, pl.ds(S - 16, 16)], S - 16, range(8, 16))

    def wait_fire(j):
        pltpu.make_async_copy(out_hbm.at[0], rows[j], gsem[j]).wait()

    def store(g, j):
        pltpu.async_copy(rows[j], out_hbm.at[base + g], ssem[j])

    def wait_store(j):
        pltpu.make_async_copy(rows[j], out_hbm.at[0], ssem[j]).wait()

    pltpu.sync_copy(idx_hbm.at[pl.ds(base, RPW)], idx_all)

    fire(0, 0)
    fire(1, 1)
    fire(2, 2)
    wait_fire(0)
    store(0, 0)
    wait_fire(1)
    store(1, 1)

    def body(t, carry):
        for j in range(3):
            g = 3 * t + j
            p = (j + 2) % 3
            wait_store(j)
            fire(g, j)
            wait_fire(p)
            store(g - 1, p)
        return carry

    lax.fori_loop(1, T_STEADY, body, 0)

    wait_store(0)
    fire(126, 0)
    wait_fire(2)
    store(125, 2)
    wait_store(1)
    fire(127, 1)
    wait_fire(0)
    store(126, 0)
    wait_fire(1)
    store(127, 1)
    wait_store(0)
    wait_store(1)
    wait_store(2)


def kernel(input_ids, weight):
    return _emb_lookup(input_ids.astype(jnp.int32), weight)
